# Initial kernel scaffold; baseline (speedup 1.0000x reference)
import jax
import jax.numpy as jnp
from jax.experimental import pallas as pl


def kernel(x, edge_index_0, edge_index_1, edge_index_2, edge_score_0, edge_score_1, edge_score_2, W0, b0, W1, b1, W2, b2, Ws, bs, gamma, beta):
    raise NotImplementedError("write your pallas kernel here")


# speedup vs baseline: 8.0882x; 8.0882x over previous
import functools

import jax
import jax.numpy as jnp
from jax import lax
from jax.experimental import pallas as pl
from jax.experimental.pallas import tpu as pltpu
from jax.experimental.pallas import tpu_sc as plsc

N = 10000
D = 128
E = 160000
R = 3

NP = 10240
NC = 2
NS = 16
NW = NC * NS
L = 16

CH = 128
ET = 16384
NCHUNK = ET // CH
EP = NW * ET
DEGP = R * NP
DEG_SL = DEGP // NS
ACC_SL = NP // NS
ZR = 64

BB = 512
NBLK = NP // BB

_f32 = jnp.float32
_i32 = jnp.int32


def _deg_body(dstg_hbm, ewp_hbm, deg_out, idx_v, val_v, zdeg_v, deg_sh):
    cid = lax.axis_index("c")
    sid = lax.axis_index("s")
    wid = sid * NC + cid

    def zbody(i, c):
        zdeg_v[pl.ds(i * L, L)] = jnp.zeros((L,), _f32)
        return c

    lax.fori_loop(0, DEG_SL // L, zbody, 0)
    pltpu.sync_copy(zdeg_v, deg_sh.at[pl.ds(sid * DEG_SL, DEG_SL)])
    plsc.subcore_barrier()

    pltpu.sync_copy(dstg_hbm.at[pl.ds(wid * NCHUNK, NCHUNK)], idx_v)
    pltpu.sync_copy(ewp_hbm.at[pl.ds(wid * NCHUNK, NCHUNK)], val_v)

    def body(j, c):
        pltpu.sync_copy(val_v.at[j], deg_sh.at[idx_v.at[j]], add=True)
        return c

    lax.fori_loop(0, NCHUNK, body, 0)
    plsc.subcore_barrier()
    pltpu.sync_copy(
        deg_sh.at[pl.ds(sid * DEG_SL, DEG_SL)],
        deg_out.at[cid, pl.ds(sid * DEG_SL, DEG_SL)],
    )


def _deg_call(dstg2, ewp2):
    mesh = plsc.VectorSubcoreMesh(core_axis_name="c", subcore_axis_name="s")
    return pl.kernel(
        _deg_body,
        out_type=jax.ShapeDtypeStruct((NC, DEGP), _f32),
        mesh=mesh,
        compiler_params=pltpu.CompilerParams(needs_layout_passes=False),
        scratch_types=[
            pltpu.VMEM((NCHUNK, CH), _i32),
            pltpu.VMEM((NCHUNK, CH), _f32),
            pltpu.VMEM((DEG_SL,), _f32),
            pltpu.VMEM_SHARED((DEGP,), _f32),
        ],
    )(dstg2, ewp2)


SBC = 8


def _msg_body(srcg_hbm, dstg_hbm, ewp_hbm, hflat_hbm, dinv_hbm, acc_out,
              sidx_v, didx_v, ew_v, dva_v, dvb_v, nrm_v, rows_v, zrow_v,
              acc_sh, semr, sema, semb):
    cid = lax.axis_index("c")
    sid = lax.axis_index("s")
    wid = sid * NC + cid

    def zbody(i, c):
        for k in range(D // L):
            zrow_v[i, pl.ds(k * L, L)] = jnp.zeros((L,), _f32)
        return c

    lax.fori_loop(0, ZR, zbody, 0)
    for t in range(ACC_SL // ZR):
        pltpu.sync_copy(
            zrow_v, acc_sh.at[pl.ds(sid * ACC_SL + t * ZR, ZR)])
    plsc.subcore_barrier()

    def superblk(sj, c):
        base = wid * NCHUNK + sj * SBC
        pltpu.sync_copy(srcg_hbm.at[pl.ds(base, SBC)], sidx_v)
        pltpu.sync_copy(dstg_hbm.at[pl.ds(base, SBC)], didx_v)
        pltpu.sync_copy(ewp_hbm.at[pl.ds(base, SBC)], ew_v)

        def chunk(j, cc):
            cpr = pltpu.make_async_copy(
                hflat_hbm.at[sidx_v.at[j]], rows_v, semr)
            cpr.start()
            cpa = pltpu.make_async_copy(
                dinv_hbm.at[sidx_v.at[j]], dva_v, sema)
            cpa.start()
            cpb = pltpu.make_async_copy(
                dinv_hbm.at[didx_v.at[j]], dvb_v, semb)
            cpb.start()
            cpa.wait()
            cpb.wait()
            for i in range(CH // L):
                sl = pl.ds(i * L, L)
                d16 = didx_v[j, sl]
                nrm_v[sl] = dva_v[sl] * ew_v[j, sl] * dvb_v[sl]
                r16 = (d16 >= NP).astype(_i32) + (d16 >= 2 * NP).astype(_i32)
                didx_v[j, sl] = d16 - r16 * NP
            cpr.wait()

            def sgroup(g, cc2):
                nv = nrm_v[pl.ds(g * L, L)]
                gbase = g * L
                for lane in range(L):
                    s = nv[lane]
                    for k in range(D // L):
                        sl = pl.ds(k * L, L)
                        rows_v[gbase + lane, sl] = rows_v[gbase + lane, sl] * s
                return cc2

            lax.fori_loop(0, CH // L, sgroup, 0)
            pltpu.sync_copy(rows_v, acc_sh.at[didx_v.at[j]], add=True)
            return cc

        lax.fori_loop(0, SBC, chunk, 0)
        return c

    lax.fori_loop(0, NCHUNK // SBC, superblk, 0)
    plsc.subcore_barrier()
    pltpu.sync_copy(
        acc_sh.at[pl.ds(sid * ACC_SL, ACC_SL)],
        acc_out.at[cid, pl.ds(sid * ACC_SL, ACC_SL)],
    )


def _msg_call(srcg2, dstg2, ewp2, hflat, dinv):
    mesh = plsc.VectorSubcoreMesh(core_axis_name="c", subcore_axis_name="s")
    return pl.kernel(
        _msg_body,
        out_type=jax.ShapeDtypeStruct((NC, NP, D), _f32),
        mesh=mesh,
        compiler_params=pltpu.CompilerParams(needs_layout_passes=False),
        scratch_types=[
            pltpu.VMEM((SBC, CH), _i32),
            pltpu.VMEM((SBC, CH), _i32),
            pltpu.VMEM((SBC, CH), _f32),
            pltpu.VMEM((CH,), _f32),
            pltpu.VMEM((CH,), _f32),
            pltpu.VMEM((CH,), _f32),
            pltpu.VMEM((CH, D), _f32),
            pltpu.VMEM((ZR, D), _f32),
            pltpu.VMEM_SHARED((NP, D), _f32),
            pltpu.SemaphoreType.DMA,
            pltpu.SemaphoreType.DMA,
            pltpu.SemaphoreType.DMA,
        ],
    )(srcg2, dstg2, ewp2, hflat, dinv)


def _dense_body(x_ref, w0, w1, w2, ws, b0, b1, b2, bs, deg0, deg1,
                h_ref, dense_ref, dinv_ref):
    deg = deg0[...] + deg1[...]
    dinv_ref[...] = lax.rsqrt(deg)
    xb = x_ref[...]
    dot = functools.partial(
        jnp.dot, preferred_element_type=_f32, precision=lax.Precision.HIGHEST)
    dense_ref[...] = dot(xb, ws[...]) + (b0[...] + b1[...] + b2[...] + bs[...])
    for r, w in enumerate((w0, w1, w2)):
        h_ref[r] = dot(xb, w[...])


def _dense_call(xp, W0, W1, W2, Ws, b0, b1, b2, bs, deg0, deg1):
    wspec = pl.BlockSpec((D, D), lambda i: (0, 0))
    bspec = pl.BlockSpec((1, D), lambda i: (0, 0))
    dspec = pl.BlockSpec((R, BB), lambda i: (0, i))
    return pl.pallas_call(
        _dense_body,
        grid=(NBLK,),
        in_specs=[
            pl.BlockSpec((BB, D), lambda i: (i, 0)),
            wspec, wspec, wspec, wspec,
            bspec, bspec, bspec, bspec,
            dspec, dspec,
        ],
        out_specs=[
            pl.BlockSpec((R, BB, D), lambda i: (0, i, 0)),
            pl.BlockSpec((BB, D), lambda i: (i, 0)),
            pl.BlockSpec((R, BB), lambda i: (0, i)),
        ],
        out_shape=[
            jax.ShapeDtypeStruct((R, NP, D), _f32),
            jax.ShapeDtypeStruct((NP, D), _f32),
            jax.ShapeDtypeStruct((R, NP), _f32),
        ],
    )(xp, W0, W1, W2, Ws, b0, b1, b2, bs, deg0, deg1)


def _bn_body(dense_ref, acc_ref, gamma_ref, beta_ref, out_ref, stat_ref):
    p = pl.program_id(0)
    i = pl.program_id(1)
    t = acc_ref[0] + acc_ref[1] + dense_ref[...]

    @pl.when(p == 0)
    def _():
        @pl.when(i == 0)
        def _():
            stat_ref[...] = jnp.zeros_like(stat_ref)

        rid = i * BB + lax.broadcasted_iota(_i32, (BB, 1), 0)
        tm = jnp.where(rid < N, t, 0.0)
        stat_ref[0:1, :] += jnp.sum(tm, axis=0, keepdims=True)
        stat_ref[1:2, :] += jnp.sum(tm * tm, axis=0, keepdims=True)

    @pl.when(p == 1)
    def _():
        mean = stat_ref[0:1, :] * (1.0 / N)
        var = stat_ref[1:2, :] * (1.0 / N) - mean * mean
        yv = (t - mean) * lax.rsqrt(var + 1e-5) * gamma_ref[...] + beta_ref[...]
        out_ref[...] = jnp.maximum(yv, 0.0)


def _bn_call(dense, accp, gamma2, beta2):
    bspec = pl.BlockSpec((1, D), lambda p, i: (0, 0))
    return pl.pallas_call(
        _bn_body,
        grid=(2, NBLK),
        in_specs=[
            pl.BlockSpec((BB, D), lambda p, i: (i, 0)),
            pl.BlockSpec((NC, BB, D), lambda p, i: (0, i, 0)),
            bspec, bspec,
        ],
        out_specs=pl.BlockSpec((BB, D), lambda p, i: (i, 0)),
        out_shape=jax.ShapeDtypeStruct((NP, D), _f32),
        scratch_shapes=[pltpu.VMEM((8, D), _f32)],
    )(dense, accp, gamma2, beta2)


def kernel(x, edge_index_0, edge_index_1, edge_index_2, edge_score_0,
           edge_score_1, edge_score_2, W0, b0, W1, b1, W2, b2, Ws, bs,
           gamma, beta):
    eis = (edge_index_0, edge_index_1, edge_index_2)
    ews = (edge_score_0, edge_score_1, edge_score_2)

    loop = jnp.arange(NP, dtype=_i32)
    srcg = jnp.concatenate(
        [eis[r][0].astype(_i32) + r * NP for r in range(R)]
        + [loop + r * NP for r in range(R)])
    dstg = jnp.concatenate(
        [eis[r][1].astype(_i32) + r * NP for r in range(R)]
        + [loop + r * NP for r in range(R)])
    ewp = jnp.concatenate(
        [ews[r] for r in range(R)] + [jnp.ones((R * NP,), _f32)])
    npad = EP - srcg.shape[0]
    srcg2 = jnp.pad(srcg, (0, npad)).reshape(EP // CH, CH)
    dstg2 = jnp.pad(dstg, (0, npad)).reshape(EP // CH, CH)
    ewp2 = jnp.pad(ewp, (0, npad)).reshape(EP // CH, CH)

    xp = jnp.pad(x, ((0, NP - N), (0, 0)))
    b0r, b1r, b2r, bsr = (b.reshape(1, D) for b in (b0, b1, b2, bs))

    degp = _deg_call(dstg2, ewp2)
    deg0 = degp[0].reshape(R, NP)
    deg1 = degp[1].reshape(R, NP)

    hflat, dense, dinv = _dense_call(
        xp, W0, W1, W2, Ws, b0r, b1r, b2r, bsr, deg0, deg1)
    hflat = hflat.reshape(DEGP, D)
    dinv = dinv.reshape(DEGP)

    accp = _msg_call(srcg2, dstg2, ewp2, hflat, dinv)

    y = _bn_call(dense, accp, gamma.reshape(1, D), beta.reshape(1, D))
    return y[:N]
